# baseline (speedup 1.0000x reference)
import functools

import jax
import jax.numpy as jnp
from jax import lax
from jax.experimental import pallas as pl
from jax.experimental.pallas import tpu as pltpu
from jax.experimental.pallas import tpu_sc as plsc

L = 16


def _sc_retile(V, D):
    info = plsc.get_sparse_core_info()
    NC, NS = info.num_cores, info.num_subcores
    NW = NC * NS
    W = 2 * D
    n_full = V // W
    tail = V - n_full * W
    mesh = plsc.VectorSubcoreMesh(core_axis_name="c", subcore_axis_name="s")

    @functools.partial(
        pl.kernel,
        mesh=mesh,
        compiler_params=pltpu.CompilerParams(needs_layout_passes=False),
        out_type=[
            jax.ShapeDtypeStruct((V // 2, W), jnp.float32),
            jax.ShapeDtypeStruct((V // 2, W), jnp.float32),
        ],
        scratch_types=[
            pltpu.VMEM((D, W), jnp.float32),
            pltpu.VMEM((D, W), jnp.float32),
            pltpu.VMEM((D, max(tail, L)), jnp.float32),
        ],
    )
    def retile(tT_hbm, cT_hbm, t_out, c_out, a_v, b_v, a2_v):
        wid = lax.axis_index("s") * NC + lax.axis_index("c")
        lane = lax.iota(jnp.int32, L)

        def make_blk_body(src, dst):
            def blk_body(b, _):
                pltpu.sync_copy(src.at[:, pl.ds(b * W, W)], a_v)
                def d_body(d, _):
                    for cg in range(W // L):
                        cvec = cg * L + lane
                        rows = cvec >> 1
                        cols = (cvec & 1) * D + d
                        plsc.store_scatter(b_v, [rows, cols],
                                           a_v[d, pl.ds(cg * L, L)])
                    return 0
                lax.fori_loop(0, D, d_body, 0)
                pltpu.sync_copy(b_v, dst.at[pl.ds(b * (W // 2), W // 2), :])
                return 0
            return blk_body

        for src, dst in ((tT_hbm, t_out), (cT_hbm, c_out)):
            blk_body = make_blk_body(src, dst)
            n_mine = (n_full - wid + NW - 1) // NW
            lax.fori_loop(0, n_mine,
                          lambda j, _, f=blk_body: f(wid + j * NW, _), 0)

        if tail:
            for t_idx, (src, dst) in enumerate(((tT_hbm, t_out),
                                                (cT_hbm, c_out))):
                @pl.when(wid == t_idx)
                def _(src=src, dst=dst):
                    pltpu.sync_copy(src.at[:, pl.ds(n_full * W, tail)], a2_v)
                    def d_body(d, _):
                        for cg in range(tail // L):
                            cvec = cg * L + lane
                            rows = cvec >> 1
                            cols = (cvec & 1) * D + d
                            plsc.store_scatter(b_v, [rows, cols],
                                               a2_v[d, pl.ds(cg * L, L)])
                        return 0
                    lax.fori_loop(0, D, d_body, 0)
                    pltpu.sync_copy(
                        b_v.at[pl.ds(0, tail // 2), :],
                        dst.at[pl.ds(n_full * (W // 2), tail // 2), :])

    return retile


def _sc_scores(B, K, D, C):
    info = plsc.get_sparse_core_info()
    NC, NS = info.num_cores, info.num_subcores
    NW = NC * NS
    assert B % (NW * C) == 0
    n_chunks = B // (NW * C)
    W = 2 * D

    mesh = plsc.VectorSubcoreMesh(core_axis_name="c", subcore_axis_name="s")

    @functools.partial(
        pl.kernel,
        mesh=mesh,
        compiler_params=pltpu.CompilerParams(needs_layout_passes=False),
        out_type=[
            jax.ShapeDtypeStruct((B,), jnp.float32),
            jax.ShapeDtypeStruct((B * K,), jnp.float32),
        ],
        scratch_types=[
            pltpu.VMEM((C,), jnp.int32),
            pltpu.VMEM((C,), jnp.int32),
            pltpu.VMEM((C * K,), jnp.int32),
            pltpu.VMEM((C,), jnp.int32),
            pltpu.VMEM((C,), jnp.int32),
            pltpu.VMEM((C * K,), jnp.int32),
            pltpu.VMEM((C,), jnp.int32),
            pltpu.VMEM((C,), jnp.int32),
            pltpu.VMEM((C * K,), jnp.int32),
            pltpu.VMEM((C, W), jnp.float32),
            pltpu.VMEM((C, W), jnp.float32),
            pltpu.VMEM((C * K, W), jnp.float32),
            pltpu.VMEM((C,), jnp.float32),
            pltpu.VMEM((C * K,), jnp.float32),
            pltpu.SemaphoreType.DMA,
            pltpu.SemaphoreType.DMA,
            pltpu.SemaphoreType.DMA,
        ],
    )
    def sc_kernel(tids_hbm, cids_hbm, nids_hbm, tW_hbm, cW_hbm,
                  pos_hbm, neg_hbm,
                  tid_v, cid_v, nid_v, tid2_v, cid2_v, nid2_v,
                  tcb_v, ccb_v, ncb_v,
                  trows, crows, nrows, posbuf, negbuf,
                  sem_t, sem_c, sem_n):
        wid = lax.axis_index("s") * NC + lax.axis_index("c")
        base = wid * (n_chunks * C)
        lane = lax.iota(jnp.int32, L)

        def halve(raw, idx2, cb, n):
            def body(g, _):
                v = raw[pl.ds(g * L, L)]
                idx2[pl.ds(g * L, L)] = v >> 1
                cb[pl.ds(g * L, L)] = (v & 1) * D
                return 0
            lax.fori_loop(0, n // L, body, 0)

        def chunk_body(ch, _):
            c0 = base + ch * C
            pltpu.sync_copy(tids_hbm.at[pl.ds(c0, C)], tid_v)
            pltpu.sync_copy(cids_hbm.at[pl.ds(c0, C)], cid_v)
            pltpu.sync_copy(nids_hbm.at[pl.ds(c0 * K, C * K)], nid_v)
            halve(tid_v, tid2_v, tcb_v, C)
            halve(cid_v, cid2_v, ccb_v, C)
            halve(nid_v, nid2_v, ncb_v, C * K)
            ct = pltpu.async_copy(tW_hbm.at[tid2_v], trows, sem_t)
            cc = pltpu.async_copy(cW_hbm.at[cid2_v], crows, sem_c)
            cn = pltpu.async_copy(cW_hbm.at[nid2_v], nrows, sem_n)
            ct.wait()
            cc.wait()
            cn.wait()

            def pos_body(g, _):
                rows = g * L + lane
                tcb = tcb_v[pl.ds(g * L, L)]
                ccb = ccb_v[pl.ds(g * L, L)]
                accs = [jnp.zeros((L,), jnp.float32) for _ in range(4)]
                for d in range(D):
                    tv = plsc.load_gather(trows, [rows, tcb + d])
                    cv = plsc.load_gather(crows, [rows, ccb + d])
                    accs[d % 4] = accs[d % 4] + tv * cv
                posbuf[pl.ds(g * L, L)] = (accs[0] + accs[1]) + (accs[2]
                                                                 + accs[3])
                return 0

            lax.fori_loop(0, C // L, pos_body, 0)

            def neg_body(g, _):
                flat = g * L + lane
                rv = lax.div(flat, K)
                ncb = ncb_v[pl.ds(g * L, L)]
                tcb = plsc.load_gather(tcb_v, [rv])
                accs = [jnp.zeros((L,), jnp.float32) for _ in range(4)]
                for d in range(D):
                    nv = plsc.load_gather(nrows, [flat, ncb + d])
                    tv = plsc.load_gather(trows, [rv, tcb + d])
                    accs[d % 4] = accs[d % 4] + nv * tv
                negbuf[pl.ds(g * L, L)] = (accs[0] + accs[1]) + (accs[2]
                                                                 + accs[3])
                return 0

            lax.fori_loop(0, C * K // L, neg_body, 0)
            pltpu.sync_copy(posbuf, pos_hbm.at[pl.ds(c0, C)])
            pltpu.sync_copy(negbuf, neg_hbm.at[pl.ds(c0 * K, C * K)])
            return 0

        lax.fori_loop(0, n_chunks, chunk_body, 0)

    return sc_kernel


def _tc_loss_kernel(pos_ref, neg_ref, out_ref):
    p = pos_ref[...]
    n = neg_ref[...]
    pos_ls = jnp.minimum(p, 0.0) - jnp.log1p(jnp.exp(-jnp.abs(p)))
    m = -n
    neg_ls = jnp.minimum(m, 0.0) - jnp.log1p(jnp.exp(-jnp.abs(m)))
    total = jnp.sum(pos_ls) + jnp.sum(neg_ls)
    out_ref[0, 0] = -total / p.size


def kernel(target_ids, context_ids, neg_ids, target_W, context_W):
    B, K = neg_ids.shape
    V, D = target_W.shape
    neg_flat = neg_ids.reshape(B * K)

    retile = _sc_retile(V, D)
    t_dense, c_dense = retile(target_W.T, context_W.T)

    sc = _sc_scores(B, K, D, C=32)
    pos_score, neg_score = sc(target_ids, context_ids, neg_flat,
                              t_dense, c_dense)

    loss = pl.pallas_call(
        _tc_loss_kernel,
        out_shape=jax.ShapeDtypeStruct((1, 1), jnp.float32),
        out_specs=pl.BlockSpec(memory_space=pltpu.SMEM),
    )(pos_score.reshape(B // 128, 128), neg_score.reshape(B * K // 128, 128))
    return loss[0, 0]


# speedup vs baseline: 2.5280x; 2.5280x over previous
import functools

import jax
import jax.numpy as jnp
from jax import lax
from jax.experimental import pallas as pl
from jax.experimental.pallas import tpu as pltpu
from jax.experimental.pallas import tpu_sc as plsc

L = 16


def _sc_compact(V, D):
    info = plsc.get_sparse_core_info()
    NC, NS = info.num_cores, info.num_subcores
    NW = NC * NS
    W = 2 * D
    NB = 40
    n_tiles = V // 8
    n_blocks = n_tiles // NB
    assert n_blocks * NB == n_tiles
    mesh = plsc.VectorSubcoreMesh(core_axis_name="c", subcore_axis_name="s")

    @functools.partial(
        pl.kernel,
        mesh=mesh,
        compiler_params=pltpu.CompilerParams(needs_layout_passes=False),
        out_type=[
            jax.ShapeDtypeStruct((V // 2, W), jnp.float32),
            jax.ShapeDtypeStruct((V // 2, W), jnp.float32),
        ],
        scratch_types=[
            pltpu.VMEM((NB, 8, D), jnp.float32),
            pltpu.VMEM((NB, 8, D), jnp.float32),
            pltpu.VMEM((NB * 4, W), jnp.float32),
            pltpu.VMEM((NB * 4, W), jnp.float32),
            pltpu.SemaphoreType.DMA,
            pltpu.SemaphoreType.DMA,
            pltpu.SemaphoreType.DMA,
            pltpu.SemaphoreType.DMA,
        ],
    )
    def compact(t3_hbm, c3_hbm, t_out, c_out,
                a0, a1, b0, b1, si0, si1, so0, so1):
        wid = lax.axis_index("s") * NC + lax.axis_index("c")
        a = (a0, a1)
        b = (b0, b1)
        si = (si0, si1)
        so = (so0, so1)

        for src, dst in ((t3_hbm, t_out), (c3_hbm, c_out)):
            n_mine = (n_blocks - wid + NW - 1) // NW

            def fire_in(j, sidx):
                q = (wid + j * NW) * NB
                return pltpu.async_copy(src.at[pl.ds(q, NB)], a[sidx],
                                        si[sidx])

            def repack(sidx):
                av, bv = a[sidx], b[sidx]

                def t_body(t, _):
                    for r in range(8):
                        for g in range(D // L):
                            bv[t * 4 + r // 2,
                               pl.ds((r % 2) * D + g * L, L)] = (
                                av[t, r, pl.ds(g * L, L)])
                    return 0

                lax.fori_loop(0, NB, t_body, 0)

            def fire_out(j, sidx):
                q = (wid + j * NW) * NB
                return pltpu.async_copy(
                    b[sidx], dst.at[pl.ds(q * 4, NB * 4), :], so[sidx])

            def phase(j, sidx):
                @pl.when(j + 1 < n_mine)
                def _():
                    fire_in(j + 1, 1 - sidx)
                pltpu.make_async_copy(
                    src.at[pl.ds(0, NB)], a[sidx], si[sidx]).wait()
                @pl.when(j >= 2)
                def _():
                    pltpu.make_async_copy(
                        b[sidx], dst.at[pl.ds(0, NB * 4), :],
                        so[sidx]).wait()
                repack(sidx)
                fire_out(j, sidx)

            fire_in(0, 0)

            def pair(pp, _):
                j0 = 2 * pp
                phase(j0, 0)
                @pl.when(j0 + 1 < n_mine)
                def _():
                    phase(j0 + 1, 1)
                return 0

            lax.fori_loop(0, (n_mine + 1) // 2, pair, 0)
            @pl.when(n_mine >= 2)
            def _():
                pltpu.make_async_copy(
                    b[0], dst.at[pl.ds(0, NB * 4), :], so[0]).wait()
                pltpu.make_async_copy(
                    b[1], dst.at[pl.ds(0, NB * 4), :], so[1]).wait()

    return compact


def _sc_scores(B, K, D, C):
    info = plsc.get_sparse_core_info()
    NC, NS = info.num_cores, info.num_subcores
    NW = NC * NS
    assert B % (NW * C) == 0 and (C * K) % L == 0
    n_chunks = B // (NW * C)
    n_sub = D // L
    W = 2 * D
    PB = B // NW
    PN = PB * K

    mesh = plsc.VectorSubcoreMesh(core_axis_name="c", subcore_axis_name="s")

    @functools.partial(
        pl.kernel,
        mesh=mesh,
        compiler_params=pltpu.CompilerParams(needs_layout_passes=False),
        out_type=[
            jax.ShapeDtypeStruct((B,), jnp.float32),
            jax.ShapeDtypeStruct((B * K,), jnp.float32),
        ],
        scratch_types=[
            pltpu.VMEM((PB + L,), jnp.int32),
            pltpu.VMEM((PB + L,), jnp.int32),
            pltpu.VMEM((PN + L,), jnp.int32),
            pltpu.VMEM((PB + L,), jnp.int32),
            pltpu.VMEM((PB + L,), jnp.int32),
            pltpu.VMEM((PN + L,), jnp.int32),
            pltpu.VMEM((L,), jnp.int32),
            pltpu.VMEM((L,), jnp.int32),
            pltpu.VMEM((L,), jnp.int32),
            pltpu.VMEM((L,), jnp.int32),
            pltpu.VMEM((C * K,), jnp.int32),
            pltpu.VMEM((C * K,), jnp.int32),
            pltpu.VMEM((L, W), jnp.float32),
            pltpu.VMEM((L, W), jnp.float32),
            pltpu.VMEM((L, W), jnp.float32),
            pltpu.VMEM((L, W), jnp.float32),
            pltpu.VMEM((C * K, W), jnp.float32),
            pltpu.VMEM((C * K, W), jnp.float32),
            pltpu.VMEM((PB,), jnp.float32),
            pltpu.VMEM((PN,), jnp.float32),
            pltpu.SemaphoreType.DMA,
            pltpu.SemaphoreType.DMA,
            pltpu.SemaphoreType.DMA,
            pltpu.SemaphoreType.DMA,
            pltpu.SemaphoreType.DMA,
            pltpu.SemaphoreType.DMA,
        ],
    )
    def sc_kernel(tids_hbm, cids_hbm, nids_hbm, tW_hbm, cW_hbm,
                  pos_hbm, neg_hbm,
                  tid_v, cid_v, nid_v, tid2_v, cid2_v, nid2_v,
                  ti0, ti1, ci0, ci1, ni0, ni1,
                  tr0, tr1, cr0, cr1, nr0, nr1,
                  posbuf, negbuf,
                  st0, st1, sc0, sc1, sn0, sn1):
        wid = lax.axis_index("s") * NC + lax.axis_index("c")
        base = wid * PB
        lane = lax.iota(jnp.int32, L)
        first = lane == 0
        ti = (ti0, ti1)
        ci = (ci0, ci1)
        ni = (ni0, ni1)
        tr = (tr0, tr1)
        cr = (cr0, cr1)
        nr = (nr0, nr1)
        st = (st0, st1)
        sc = (sc0, sc1)
        sn = (sn0, sn1)

        dnums = lax.GatherDimensionNumbers(
            offset_dims=(), collapsed_slice_dims=(0,), start_index_map=(0,))

        def shuffle(v, idx):
            return lax.gather(v, idx[:, None], dimension_numbers=dnums,
                              slice_sizes=(1,),
                              mode=lax.GatherScatterMode.PROMISE_IN_BOUNDS)

        def hsum(v):
            for s in (8, 4, 2, 1):
                v = v + shuffle(v, lane ^ s)
            return v

        pltpu.sync_copy(tids_hbm.at[pl.ds(base, PB)], tid_v.at[pl.ds(0, PB)])
        pltpu.sync_copy(cids_hbm.at[pl.ds(base, PB)], cid_v.at[pl.ds(0, PB)])
        pltpu.sync_copy(nids_hbm.at[pl.ds(base * K, PN)],
                        nid_v.at[pl.ds(0, PN)])

        def halve(raw, idx2, n):
            def body(g, _):
                idx2[pl.ds(g * L, L)] = raw[pl.ds(g * L, L)] >> 1
                return 0
            lax.fori_loop(0, n // L, body, 0)

        halve(tid_v, tid2_v, PB)
        halve(cid_v, cid2_v, PB)
        halve(nid_v, nid2_v, PN)
        tid2_v[pl.ds(PB, L)] = lane * 0
        cid2_v[pl.ds(PB, L)] = lane * 0

        def fire(ch, s):
            ti[s][...] = tid2_v[pl.ds(ch * C, L)]
            ci[s][...] = cid2_v[pl.ds(ch * C, L)]
            for g in range(C * K // L):
                ni[s][pl.ds(g * L, L)] = nid2_v[pl.ds(ch * C * K + g * L, L)]
            pltpu.async_copy(tW_hbm.at[ti[s]], tr[s], st[s])
            pltpu.async_copy(cW_hbm.at[ci[s]], cr[s], sc[s])
            pltpu.async_copy(cW_hbm.at[ni[s]], nr[s], sn[s])

        def wait(s):
            pltpu.make_async_copy(tW_hbm.at[ti[s]], tr[s], st[s]).wait()
            pltpu.make_async_copy(cW_hbm.at[ci[s]], cr[s], sc[s]).wait()
            pltpu.make_async_copy(cW_hbm.at[ni[s]], nr[s], sn[s]).wait()

        def compute(ch, s):
            trows, crows, nrows = tr[s], cr[s], nr[s]
            traw = tid_v[pl.ds(ch * C, L)]
            craw = cid_v[pl.ds(ch * C, L)]
            nraws = [nid_v[pl.ds(ch * C * K + g * L, L)]
                     for g in range(C * K // L)]
            for i in range(C):
                tb = (traw[i] & 1) * D
                cb = (craw[i] & 1) * D
                t = [trows[i, pl.ds(tb + j * L, L)] for j in range(n_sub)]
                acc = t[0] * crows[i, pl.ds(cb, L)]
                for j in range(1, n_sub):
                    acc = acc + t[j] * crows[i, pl.ds(cb + j * L, L)]
                plsc.store_scatter(posbuf, [lane * 0 + (ch * C + i)],
                                   hsum(acc), mask=first)
                for k in range(K):
                    r = i * K + k
                    nb = (nraws[r // L][r % L] & 1) * D
                    acc = t[0] * nrows[r, pl.ds(nb, L)]
                    for j in range(1, n_sub):
                        acc = acc + t[j] * nrows[r, pl.ds(nb + j * L, L)]
                    plsc.store_scatter(negbuf, [lane * 0 + (ch * C * K + r)],
                                       hsum(acc), mask=first)

        fire(0, 0)

        def pair(pp, _):
            ch0 = 2 * pp
            fire(ch0 + 1, 1)
            wait(0)
            compute(ch0, 0)
            @pl.when(ch0 + 2 < n_chunks)
            def _():
                fire(ch0 + 2, 0)
            wait(1)
            compute(ch0 + 1, 1)
            return 0

        lax.fori_loop(0, n_chunks // 2, pair, 0)

        pltpu.sync_copy(posbuf, pos_hbm.at[pl.ds(base, PB)])
        pltpu.sync_copy(negbuf, neg_hbm.at[pl.ds(base * K, PN)])

    return sc_kernel


def _tc_loss_kernel(pos_ref, neg_ref, out_ref):
    p = pos_ref[...]
    n = neg_ref[...]
    pos_ls = jnp.minimum(p, 0.0) - jnp.log1p(jnp.exp(-jnp.abs(p)))
    m = -n
    neg_ls = jnp.minimum(m, 0.0) - jnp.log1p(jnp.exp(-jnp.abs(m)))
    total = jnp.sum(pos_ls) + jnp.sum(neg_ls)
    out_ref[0, 0] = -total / p.size


def kernel(target_ids, context_ids, neg_ids, target_W, context_W):
    B, K = neg_ids.shape
    V, D = target_W.shape
    neg_flat = neg_ids.reshape(B * K)

    compact = _sc_compact(V, D)
    t_dense, c_dense = compact(target_W.reshape(V // 8, 8, D),
                               context_W.reshape(V // 8, 8, D))

    sc = _sc_scores(B, K, D, C=8)
    pos_score, neg_score = sc(target_ids, context_ids, neg_flat,
                              t_dense, c_dense)

    loss = pl.pallas_call(
        _tc_loss_kernel,
        out_shape=jax.ShapeDtypeStruct((1, 1), jnp.float32),
        out_specs=pl.BlockSpec(memory_space=pltpu.SMEM),
    )(pos_score.reshape(B // 128, 128), neg_score.reshape(B * K // 128, 128))
    return loss[0, 0]


# speedup vs baseline: 4.3037x; 1.7024x over previous
import functools

import jax
import jax.numpy as jnp
from jax import lax
from jax.experimental import pallas as pl
from jax.experimental.pallas import tpu as pltpu
from jax.experimental.pallas import tpu_sc as plsc

L = 16


def _sc_scores(B, K, D, C):
    info = plsc.get_sparse_core_info()
    NC, NS = info.num_cores, info.num_subcores
    NW = NC * NS
    assert B % (NW * C) == 0
    n_chunks = B // (NW * C)
    assert n_chunks % 2 == 0
    n_sub = D // L
    PB = B // NW
    PN = PB * K
    CN = C * K

    mesh = plsc.VectorSubcoreMesh(core_axis_name="c", subcore_axis_name="s")

    @functools.partial(
        pl.kernel,
        mesh=mesh,
        compiler_params=pltpu.CompilerParams(needs_layout_passes=False),
        out_type=[
            jax.ShapeDtypeStruct((B,), jnp.float32),
            jax.ShapeDtypeStruct((B * K,), jnp.float32),
        ],
        scratch_types=[
            pltpu.VMEM((PB + L,), jnp.int32),
            pltpu.VMEM((PB + L,), jnp.int32),
            pltpu.VMEM((PN + L,), jnp.int32),
            pltpu.VMEM((C, 8, D), jnp.float32),
            pltpu.VMEM((C, 8, D), jnp.float32),
            pltpu.VMEM((C, 8, D), jnp.float32),
            pltpu.VMEM((C, 8, D), jnp.float32),
            pltpu.VMEM((C * K, 8, D), jnp.float32),
            pltpu.VMEM((C * K, 8, D), jnp.float32),
            pltpu.VMEM((PB,), jnp.float32),
            pltpu.VMEM((PN,), jnp.float32),
            pltpu.SemaphoreType.DMA,
            pltpu.SemaphoreType.DMA,
            pltpu.SemaphoreType.DMA,
            pltpu.SemaphoreType.DMA,
            pltpu.SemaphoreType.DMA,
            pltpu.SemaphoreType.DMA,
        ],
    )
    def sc_kernel(tids_hbm, cids_hbm, nids_hbm, tW_hbm, cW_hbm,
                  pos_hbm, neg_hbm,
                  tid_v, cid_v, nid_v,
                  rt0, rt1, rc0, rc1, rn0, rn1,
                  posbuf, negbuf,
                  st0, st1, sc0, sc1, sn0, sn1):
        wid = lax.axis_index("s") * NC + lax.axis_index("c")
        base = wid * PB
        lane = lax.iota(jnp.int32, L)
        first = lane == 0
        rt = (rt0, rt1)
        rc = (rc0, rc1)
        rn = (rn0, rn1)
        st = (st0, st1)
        sc = (sc0, sc1)
        sn = (sn0, sn1)

        dnums = lax.GatherDimensionNumbers(
            offset_dims=(), collapsed_slice_dims=(0,), start_index_map=(0,))

        def shuffle(v, idx):
            return lax.gather(v, idx[:, None], dimension_numbers=dnums,
                              slice_sizes=(1,),
                              mode=lax.GatherScatterMode.PROMISE_IN_BOUNDS)

        def hsum(v):
            for s in (8, 4, 2, 1):
                v = v + shuffle(v, lane ^ s)
            return v

        pltpu.sync_copy(tids_hbm.at[pl.ds(base, PB)], tid_v.at[pl.ds(0, PB)])
        pltpu.sync_copy(cids_hbm.at[pl.ds(base, PB)], cid_v.at[pl.ds(0, PB)])
        pltpu.sync_copy(nids_hbm.at[pl.ds(base * K, PN)],
                        nid_v.at[pl.ds(0, PN)])
        tid_v[pl.ds(PB, L)] = lane * 0
        cid_v[pl.ds(PB, L)] = lane * 0
        nid_v[pl.ds(PN, L)] = lane * 0

        def id_vecs(ch):
            traw = tid_v[pl.ds(ch * C, L)]
            craw = cid_v[pl.ds(ch * C, L)]
            nraws = [nid_v[pl.ds(ch * CN + g * L, L)]
                     for g in range((CN + L - 1) // L)]
            return traw, craw, nraws

        def nid_at(nraws, r):
            return nraws[r // L][r % L]

        def fire(ch, s):
            traw, craw, nraws = id_vecs(ch)
            for i in range(C):
                pltpu.async_copy(tW_hbm.at[traw[i] >> 3], rt[s].at[i], st[s])
                pltpu.async_copy(cW_hbm.at[craw[i] >> 3], rc[s].at[i], sc[s])
            for r in range(CN):
                pltpu.async_copy(cW_hbm.at[nid_at(nraws, r) >> 3],
                                 rn[s].at[r], sn[s])

        def wait(ch, s):
            traw, craw, nraws = id_vecs(ch)
            for i in range(C):
                pltpu.make_async_copy(tW_hbm.at[traw[i] >> 3],
                                      rt[s].at[i], st[s]).wait()
                pltpu.make_async_copy(cW_hbm.at[craw[i] >> 3],
                                      rc[s].at[i], sc[s]).wait()
            for r in range(CN):
                pltpu.make_async_copy(cW_hbm.at[nid_at(nraws, r) >> 3],
                                      rn[s].at[r], sn[s]).wait()

        def compute(ch, s):
            traw, craw, nraws = id_vecs(ch)
            for i in range(C):
                trow = traw[i] & 7
                crow = craw[i] & 7
                t = [rt[s][i, trow, pl.ds(j * L, L)] for j in range(n_sub)]
                acc = t[0] * rc[s][i, crow, pl.ds(0, L)]
                for j in range(1, n_sub):
                    acc = acc + t[j] * rc[s][i, crow, pl.ds(j * L, L)]
                plsc.store_scatter(posbuf, [lane * 0 + (ch * C + i)],
                                   hsum(acc), mask=first)
                for k in range(K):
                    r = i * K + k
                    nrow = nid_at(nraws, r) & 7
                    acc = t[0] * rn[s][r, nrow, pl.ds(0, L)]
                    for j in range(1, n_sub):
                        acc = acc + t[j] * rn[s][r, nrow, pl.ds(j * L, L)]
                    plsc.store_scatter(negbuf, [lane * 0 + (ch * CN + r)],
                                       hsum(acc), mask=first)

        fire(0, 0)

        def pair(pp, _):
            ch0 = 2 * pp
            fire(ch0 + 1, 1)
            wait(ch0, 0)
            compute(ch0, 0)
            @pl.when(ch0 + 2 < n_chunks)
            def _():
                fire(ch0 + 2, 0)
            wait(ch0 + 1, 1)
            compute(ch0 + 1, 1)
            return 0

        lax.fori_loop(0, n_chunks // 2, pair, 0)

        pltpu.sync_copy(posbuf, pos_hbm.at[pl.ds(base, PB)])
        pltpu.sync_copy(negbuf, neg_hbm.at[pl.ds(base * K, PN)])

    return sc_kernel


def _tc_loss_kernel(pos_ref, neg_ref, out_ref):
    p = pos_ref[...]
    n = neg_ref[...]
    pos_ls = jnp.minimum(p, 0.0) - jnp.log1p(jnp.exp(-jnp.abs(p)))
    m = -n
    neg_ls = jnp.minimum(m, 0.0) - jnp.log1p(jnp.exp(-jnp.abs(m)))
    total = jnp.sum(pos_ls) + jnp.sum(neg_ls)
    out_ref[0, 0] = -total / p.size


def kernel(target_ids, context_ids, neg_ids, target_W, context_W):
    B, K = neg_ids.shape
    V, D = target_W.shape
    neg_flat = neg_ids.reshape(B * K)

    sc = _sc_scores(B, K, D, C=2)
    pos_score, neg_score = sc(target_ids, context_ids, neg_flat,
                              target_W.reshape(V // 8, 8, D),
                              context_W.reshape(V // 8, 8, D))

    loss = pl.pallas_call(
        _tc_loss_kernel,
        out_shape=jax.ShapeDtypeStruct((1, 1), jnp.float32),
        out_specs=pl.BlockSpec(memory_space=pltpu.SMEM),
    )(pos_score.reshape(B // 128, 128), neg_score.reshape(B * K // 128, 128))
    return loss[0, 0]


# speedup vs baseline: 4.6463x; 1.0796x over previous
import functools

import jax
import jax.numpy as jnp
from jax import lax
from jax.experimental import pallas as pl
from jax.experimental.pallas import tpu as pltpu
from jax.experimental.pallas import tpu_sc as plsc

L = 16


def _sc_scores(B, K, D, C):
    info = plsc.get_sparse_core_info()
    NC, NS = info.num_cores, info.num_subcores
    NW = NC * NS
    assert B % (NW * C) == 0
    n_chunks = B // (NW * C)
    assert n_chunks % 4 == 0
    n_sub = D // L
    PB = B // NW
    PN = PB * K
    CN = C * K

    mesh = plsc.VectorSubcoreMesh(core_axis_name="c", subcore_axis_name="s")

    @functools.partial(
        pl.kernel,
        mesh=mesh,
        compiler_params=pltpu.CompilerParams(needs_layout_passes=False),
        out_type=[
            jax.ShapeDtypeStruct((B,), jnp.float32),
            jax.ShapeDtypeStruct((B * K,), jnp.float32),
        ],
        scratch_types=[
            pltpu.VMEM((PB + L,), jnp.int32),
            pltpu.VMEM((PB + L,), jnp.int32),
            pltpu.VMEM((PN + L,), jnp.int32),
        ] + [pltpu.VMEM((C, 8, D), jnp.float32) for _ in range(4)]
          + [pltpu.VMEM((C, 8, D), jnp.float32) for _ in range(4)]
          + [pltpu.VMEM((C * K, 8, D), jnp.float32) for _ in range(4)]
          + [
            pltpu.VMEM((PB,), jnp.float32),
            pltpu.VMEM((PN,), jnp.float32),
        ] + [pltpu.SemaphoreType.DMA for _ in range(12)],
    )
    def sc_kernel(tids_hbm, cids_hbm, nids_hbm, tW_hbm, cW_hbm,
                  pos_hbm, neg_hbm,
                  tid_v, cid_v, nid_v,
                  rt0, rt1, rt2, rt3, rc0, rc1, rc2, rc3,
                  rn0, rn1, rn2, rn3,
                  posbuf, negbuf,
                  st0, st1, st2, st3, sc0, sc1, sc2, sc3,
                  sn0, sn1, sn2, sn3):
        wid = lax.axis_index("s") * NC + lax.axis_index("c")
        base = wid * PB
        lane = lax.iota(jnp.int32, L)
        first = lane == 0
        rt = (rt0, rt1, rt2, rt3)
        rc = (rc0, rc1, rc2, rc3)
        rn = (rn0, rn1, rn2, rn3)
        st = (st0, st1, st2, st3)
        sc = (sc0, sc1, sc2, sc3)
        sn = (sn0, sn1, sn2, sn3)

        dnums = lax.GatherDimensionNumbers(
            offset_dims=(), collapsed_slice_dims=(0,), start_index_map=(0,))

        def shuffle(v, idx):
            return lax.gather(v, idx[:, None], dimension_numbers=dnums,
                              slice_sizes=(1,),
                              mode=lax.GatherScatterMode.PROMISE_IN_BOUNDS)

        def hsum(v):
            for s in (8, 4, 2, 1):
                v = v + shuffle(v, lane ^ s)
            return v

        pltpu.sync_copy(tids_hbm.at[pl.ds(base, PB)], tid_v.at[pl.ds(0, PB)])
        pltpu.sync_copy(cids_hbm.at[pl.ds(base, PB)], cid_v.at[pl.ds(0, PB)])
        pltpu.sync_copy(nids_hbm.at[pl.ds(base * K, PN)],
                        nid_v.at[pl.ds(0, PN)])
        tid_v[pl.ds(PB, L)] = lane * 0
        cid_v[pl.ds(PB, L)] = lane * 0
        nid_v[pl.ds(PN, L)] = lane * 0

        def id_vecs(ch):
            traw = tid_v[pl.ds(ch * C, L)]
            craw = cid_v[pl.ds(ch * C, L)]
            nraws = [nid_v[pl.ds(ch * CN + g * L, L)]
                     for g in range((CN + L - 1) // L)]
            return traw, craw, nraws

        def nid_at(nraws, r):
            return nraws[r // L][r % L]

        def fire(ch, s):
            traw, craw, nraws = id_vecs(ch)
            for i in range(C):
                pltpu.async_copy(tW_hbm.at[traw[i] >> 3], rt[s].at[i], st[s])
                pltpu.async_copy(cW_hbm.at[craw[i] >> 3], rc[s].at[i], sc[s])
            for r in range(CN):
                pltpu.async_copy(cW_hbm.at[nid_at(nraws, r) >> 3],
                                 rn[s].at[r], sn[s])

        def wait(ch, s):
            pltpu.make_async_copy(tW_hbm.at[pl.ds(0, C)], rt[s],
                                  st[s]).wait()
            pltpu.make_async_copy(cW_hbm.at[pl.ds(0, C)], rc[s],
                                  sc[s]).wait()
            pltpu.make_async_copy(cW_hbm.at[pl.ds(0, CN)], rn[s],
                                  sn[s]).wait()

        zero16 = lane * 0

        def compute(ch, s):
            traw, craw, nraws = id_vecs(ch)
            for i in range(C):
                trow = traw[i] & 7
                crow = craw[i] & 7
                t = [rt[s][i, trow, pl.ds(j * L, L)] for j in range(n_sub)]
                acc = t[0] * rc[s][i, crow, pl.ds(0, L)]
                for j in range(1, n_sub):
                    acc = acc + t[j] * rc[s][i, crow, pl.ds(j * L, L)]
                plsc.store_scatter(posbuf, [lane * 0 + (ch * C + i)],
                                   hsum(acc), mask=first)
                for k in range(K):
                    r = i * K + k
                    nrow = nid_at(nraws, r) & 7
                    acc = t[0] * rn[s][r, nrow, pl.ds(0, L)]
                    for j in range(1, n_sub):
                        acc = acc + t[j] * rn[s][r, nrow, pl.ds(j * L, L)]
                    plsc.store_scatter(negbuf, [lane * 0 + (ch * CN + r)],
                                       hsum(acc), mask=first)

        fire(0, 0)
        fire(1, 1)
        fire(2, 2)

        def quad(qq, _):
            ch0 = 4 * qq
            for p in range(4):
                @pl.when(ch0 + p + 3 < n_chunks)
                def _(p=p):
                    fire(ch0 + p + 3, (p + 3) % 4)
                wait(ch0 + p, p)
                compute(ch0 + p, p)
            return 0

        lax.fori_loop(0, n_chunks // 4, quad, 0)

        pltpu.sync_copy(posbuf, pos_hbm.at[pl.ds(base, PB)])
        pltpu.sync_copy(negbuf, neg_hbm.at[pl.ds(base * K, PN)])

    return sc_kernel


def _tc_loss_kernel(pos_ref, neg_ref, out_ref):
    p = pos_ref[...]
    n = neg_ref[...]
    pos_ls = jnp.minimum(p, 0.0) - jnp.log1p(jnp.exp(-jnp.abs(p)))
    m = -n
    neg_ls = jnp.minimum(m, 0.0) - jnp.log1p(jnp.exp(-jnp.abs(m)))
    total = jnp.sum(pos_ls) + jnp.sum(neg_ls)
    out_ref[0, 0] = -total / p.size


def kernel(target_ids, context_ids, neg_ids, target_W, context_W):
    B, K = neg_ids.shape
    V, D = target_W.shape
    neg_flat = neg_ids.reshape(B * K)

    sc = _sc_scores(B, K, D, C=1)
    pos_score, neg_score = sc(target_ids, context_ids, neg_flat,
                              target_W.reshape(V // 8, 8, D),
                              context_W.reshape(V // 8, 8, D))

    loss = pl.pallas_call(
        _tc_loss_kernel,
        out_shape=jax.ShapeDtypeStruct((1, 1), jnp.float32),
        out_specs=pl.BlockSpec(memory_space=pltpu.SMEM),
    )(pos_score.reshape(B // 128, 128), neg_score.reshape(B * K // 128, 128))
    return loss[0, 0]


# speedup vs baseline: 4.6771x; 1.0066x over previous
import functools

import jax
import jax.numpy as jnp
from jax import lax
from jax.experimental import pallas as pl
from jax.experimental.pallas import tpu as pltpu
from jax.experimental.pallas import tpu_sc as plsc

L = 16


def _sc_scores(B, K, D, C):
    info = plsc.get_sparse_core_info()
    NC, NS = info.num_cores, info.num_subcores
    NW = NC * NS
    assert B % (NW * C) == 0
    n_chunks = B // (NW * C)
    assert n_chunks % 4 == 0
    n_sub = D // L
    PB = B // NW
    PN = PB * K
    CN = C * K

    mesh = plsc.VectorSubcoreMesh(core_axis_name="c", subcore_axis_name="s")

    @functools.partial(
        pl.kernel,
        mesh=mesh,
        compiler_params=pltpu.CompilerParams(needs_layout_passes=False),
        out_type=[
            jax.ShapeDtypeStruct((B,), jnp.float32),
            jax.ShapeDtypeStruct((B * K,), jnp.float32),
        ],
        scratch_types=[
            pltpu.VMEM((PB + L,), jnp.int32),
            pltpu.VMEM((PB + L,), jnp.int32),
            pltpu.VMEM((PN + L,), jnp.int32),
        ] + [pltpu.VMEM((C, 8, D), jnp.float32) for _ in range(4)]
          + [pltpu.VMEM((C, 8, D), jnp.float32) for _ in range(4)]
          + [pltpu.VMEM((C * K, 8, D), jnp.float32) for _ in range(4)]
          + [
            pltpu.VMEM((PB,), jnp.float32),
            pltpu.VMEM((PN,), jnp.float32),
        ] + [pltpu.SemaphoreType.DMA for _ in range(12)],
    )
    def sc_kernel(tids_hbm, cids_hbm, nids_hbm, tW_hbm, cW_hbm,
                  pos_hbm, neg_hbm,
                  tid_v, cid_v, nid_v,
                  rt0, rt1, rt2, rt3, rc0, rc1, rc2, rc3,
                  rn0, rn1, rn2, rn3,
                  posbuf, negbuf,
                  st0, st1, st2, st3, sc0, sc1, sc2, sc3,
                  sn0, sn1, sn2, sn3):
        wid = lax.axis_index("s") * NC + lax.axis_index("c")
        base = wid * PB
        lane = lax.iota(jnp.int32, L)
        first = lane == 0
        rt = (rt0, rt1, rt2, rt3)
        rc = (rc0, rc1, rc2, rc3)
        rn = (rn0, rn1, rn2, rn3)
        st = (st0, st1, st2, st3)
        sc = (sc0, sc1, sc2, sc3)
        sn = (sn0, sn1, sn2, sn3)

        dnums = lax.GatherDimensionNumbers(
            offset_dims=(), collapsed_slice_dims=(0,), start_index_map=(0,))

        def shuffle(v, idx):
            return lax.gather(v, idx[:, None], dimension_numbers=dnums,
                              slice_sizes=(1,),
                              mode=lax.GatherScatterMode.PROMISE_IN_BOUNDS)

        def hsum(v):
            for s in (8, 4, 2, 1):
                v = v + shuffle(v, lane ^ s)
            return v

        pltpu.sync_copy(tids_hbm.at[pl.ds(base, PB)], tid_v.at[pl.ds(0, PB)])
        pltpu.sync_copy(cids_hbm.at[pl.ds(base, PB)], cid_v.at[pl.ds(0, PB)])
        pltpu.sync_copy(nids_hbm.at[pl.ds(base * K, PN)],
                        nid_v.at[pl.ds(0, PN)])
        tid_v[pl.ds(PB, L)] = lane * 0
        cid_v[pl.ds(PB, L)] = lane * 0
        nid_v[pl.ds(PN, L)] = lane * 0

        def id_vecs(ch):
            traw = tid_v[pl.ds(ch * C, L)]
            craw = cid_v[pl.ds(ch * C, L)]
            nraws = [nid_v[pl.ds(ch * CN + g * L, L)]
                     for g in range((CN + L - 1) // L)]
            return traw, craw, nraws

        def nid_at(nraws, r):
            return nraws[r // L][r % L]

        def fire(ch, s):
            traw, craw, nraws = id_vecs(ch)
            for r in range(CN):
                pltpu.async_copy(cW_hbm.at[nid_at(nraws, r) >> 3],
                                 rn[s].at[r], sn[s])
            for i in range(C):
                pltpu.async_copy(tW_hbm.at[traw[i] >> 3], rt[s].at[i], st[s])
                pltpu.async_copy(cW_hbm.at[craw[i] >> 3], rc[s].at[i], sc[s])

        def compute(ch, s):
            traw, craw, nraws = id_vecs(ch)
            pltpu.make_async_copy(tW_hbm.at[pl.ds(0, C)], rt[s],
                                  st[s]).wait()
            pltpu.make_async_copy(cW_hbm.at[pl.ds(0, C)], rc[s],
                                  sc[s]).wait()
            ts = []
            for i in range(C):
                trow = traw[i] & 7
                crow = craw[i] & 7
                t = [rt[s][i, trow, pl.ds(j * L, L)] for j in range(n_sub)]
                ts.append(t)
                acc = t[0] * rc[s][i, crow, pl.ds(0, L)]
                for j in range(1, n_sub):
                    acc = acc + t[j] * rc[s][i, crow, pl.ds(j * L, L)]
                plsc.store_scatter(posbuf, [lane * 0 + (ch * C + i)],
                                   hsum(acc), mask=first)
            pltpu.make_async_copy(cW_hbm.at[pl.ds(0, CN)], rn[s],
                                  sn[s]).wait()
            for i in range(C):
                t = ts[i]
                for k in range(K):
                    r = i * K + k
                    nrow = nid_at(nraws, r) & 7
                    acc = t[0] * rn[s][r, nrow, pl.ds(0, L)]
                    for j in range(1, n_sub):
                        acc = acc + t[j] * rn[s][r, nrow, pl.ds(j * L, L)]
                    plsc.store_scatter(negbuf, [lane * 0 + (ch * CN + r)],
                                       hsum(acc), mask=first)

        fire(0, 0)
        fire(1, 1)
        fire(2, 2)

        def quad(qq, _):
            ch0 = 4 * qq
            for p in range(4):
                @pl.when(ch0 + p + 3 < n_chunks)
                def _(p=p):
                    fire(ch0 + p + 3, (p + 3) % 4)
                compute(ch0 + p, p)
            return 0

        lax.fori_loop(0, n_chunks // 4, quad, 0)

        pltpu.sync_copy(posbuf, pos_hbm.at[pl.ds(base, PB)])
        pltpu.sync_copy(negbuf, neg_hbm.at[pl.ds(base * K, PN)])

    return sc_kernel


def _tc_loss_kernel(pos_ref, neg_ref, out_ref):
    p = pos_ref[...]
    n = neg_ref[...]
    pos_ls = jnp.minimum(p, 0.0) - jnp.log1p(jnp.exp(-jnp.abs(p)))
    m = -n
    neg_ls = jnp.minimum(m, 0.0) - jnp.log1p(jnp.exp(-jnp.abs(m)))
    total = jnp.sum(pos_ls) + jnp.sum(neg_ls)
    out_ref[0, 0] = -total / p.size


def kernel(target_ids, context_ids, neg_ids, target_W, context_W):
    B, K = neg_ids.shape
    V, D = target_W.shape
    neg_flat = neg_ids.reshape(B * K)

    sc = _sc_scores(B, K, D, C=1)
    pos_score, neg_score = sc(target_ids, context_ids, neg_flat,
                              target_W.reshape(V // 8, 8, D),
                              context_W.reshape(V // 8, 8, D))

    loss = pl.pallas_call(
        _tc_loss_kernel,
        out_shape=jax.ShapeDtypeStruct((1, 1), jnp.float32),
        out_specs=pl.BlockSpec(memory_space=pltpu.SMEM),
    )(pos_score.reshape(B // 128, 128), neg_score.reshape(B * K // 128, 128))
    return loss[0, 0]


# speedup vs baseline: 4.7647x; 1.0187x over previous
import functools

import jax
import jax.numpy as jnp
from jax import lax
from jax.experimental import pallas as pl
from jax.experimental.pallas import tpu as pltpu
from jax.experimental.pallas import tpu_sc as plsc

L = 16


def _sc_scores(B, K, D, C):
    info = plsc.get_sparse_core_info()
    NC, NS = info.num_cores, info.num_subcores
    NW = NC * NS
    assert B % (NW * C) == 0
    n_chunks = B // (NW * C)
    assert n_chunks % 4 == 0
    n_sub = D // L
    PB = B // NW
    PN = PB * K
    CN = C * K

    mesh = plsc.VectorSubcoreMesh(core_axis_name="c", subcore_axis_name="s")

    @functools.partial(
        pl.kernel,
        mesh=mesh,
        compiler_params=pltpu.CompilerParams(needs_layout_passes=False),
        out_type=[
            jax.ShapeDtypeStruct((B,), jnp.float32),
            jax.ShapeDtypeStruct((B * K,), jnp.float32),
        ],
        scratch_types=[
            pltpu.VMEM((PB + L,), jnp.int32),
            pltpu.VMEM((PB + L,), jnp.int32),
            pltpu.VMEM((PN + L,), jnp.int32),
        ] + [pltpu.VMEM((C, 8, D), jnp.float32) for _ in range(4)]
          + [pltpu.VMEM((C, 8, D), jnp.float32) for _ in range(4)]
          + [pltpu.VMEM((C * K, 8, D), jnp.float32) for _ in range(4)]
          + [
            pltpu.VMEM((PB,), jnp.float32),
            pltpu.VMEM((PN,), jnp.float32),
        ] + [pltpu.SemaphoreType.DMA for _ in range(12)],
    )
    def sc_kernel(tids_hbm, cids_hbm, nids_hbm, tW_hbm, cW_hbm,
                  pos_hbm, neg_hbm,
                  tid_v, cid_v, nid_v,
                  rt0, rt1, rt2, rt3, rc0, rc1, rc2, rc3,
                  rn0, rn1, rn2, rn3,
                  posbuf, negbuf,
                  st0, st1, st2, st3, sc0, sc1, sc2, sc3,
                  sn0, sn1, sn2, sn3):
        wid = lax.axis_index("s") * NC + lax.axis_index("c")
        base = wid * PB
        lane = lax.iota(jnp.int32, L)
        first = lane == 0
        rt = (rt0, rt1, rt2, rt3)
        rc = (rc0, rc1, rc2, rc3)
        rn = (rn0, rn1, rn2, rn3)
        st = (st0, st1, st2, st3)
        sc = (sc0, sc1, sc2, sc3)
        sn = (sn0, sn1, sn2, sn3)

        dnums = lax.GatherDimensionNumbers(
            offset_dims=(), collapsed_slice_dims=(0,), start_index_map=(0,))

        def shuffle(v, idx):
            return lax.gather(v, idx[:, None], dimension_numbers=dnums,
                              slice_sizes=(1,),
                              mode=lax.GatherScatterMode.PROMISE_IN_BOUNDS)

        def hsum(v):
            for s in (8, 4, 2, 1):
                v = v + shuffle(v, lane ^ s)
            return v

        def hsum2(va, vb):
            m1 = va + shuffle(va, lane ^ 8)
            m2 = vb + shuffle(vb, lane ^ 8)
            m = jnp.where(lane < 8, m1, m2)
            for s in (4, 2, 1):
                m = m + shuffle(m, lane ^ s)
            return m

        pltpu.sync_copy(tids_hbm.at[pl.ds(base, PB)], tid_v.at[pl.ds(0, PB)])
        pltpu.sync_copy(cids_hbm.at[pl.ds(base, PB)], cid_v.at[pl.ds(0, PB)])
        pltpu.sync_copy(nids_hbm.at[pl.ds(base * K, PN)],
                        nid_v.at[pl.ds(0, PN)])
        tid_v[pl.ds(PB, L)] = lane * 0
        cid_v[pl.ds(PB, L)] = lane * 0
        nid_v[pl.ds(PN, L)] = lane * 0

        def id_vecs(ch):
            traw = tid_v[pl.ds(ch * C, L)]
            craw = cid_v[pl.ds(ch * C, L)]
            nraws = [nid_v[pl.ds(ch * CN + g * L, L)]
                     for g in range((CN + L - 1) // L)]
            return traw, craw, nraws

        def nid_at(nraws, r):
            return nraws[r // L][r % L]

        def fire(ch, s):
            traw, craw, nraws = id_vecs(ch)
            for r in range(CN):
                pltpu.async_copy(cW_hbm.at[nid_at(nraws, r) >> 3],
                                 rn[s].at[r], sn[s])
            for i in range(C):
                pltpu.async_copy(tW_hbm.at[traw[i] >> 3], rt[s].at[i], st[s])
                pltpu.async_copy(cW_hbm.at[craw[i] >> 3], rc[s].at[i], sc[s])

        def compute(ch, s):
            traw, craw, nraws = id_vecs(ch)
            pltpu.make_async_copy(tW_hbm.at[pl.ds(0, C)], rt[s],
                                  st[s]).wait()
            pltpu.make_async_copy(cW_hbm.at[pl.ds(0, C)], rc[s],
                                  sc[s]).wait()
            ts = []
            for i in range(C):
                trow = traw[i] & 7
                crow = craw[i] & 7
                t = [rt[s][i, trow, pl.ds(j * L, L)] for j in range(n_sub)]
                ts.append(t)
                acc = t[0] * rc[s][i, crow, pl.ds(0, L)]
                for j in range(1, n_sub):
                    acc = acc + t[j] * rc[s][i, crow, pl.ds(j * L, L)]
                plsc.store_scatter(posbuf, [lane * 0 + (ch * C + i)],
                                   hsum(acc), mask=first)
            pltpu.make_async_copy(cW_hbm.at[pl.ds(0, CN)], rn[s],
                                  sn[s]).wait()
            lane07 = (lane & 7) == 0
            for i in range(C):
                t = ts[i]
                for k in range(0, K, 2):
                    ra = i * K + k
                    rb = ra + 1
                    nrow_a = nid_at(nraws, ra) & 7
                    nrow_b = nid_at(nraws, rb) & 7
                    acc_a = t[0] * rn[s][ra, nrow_a, pl.ds(0, L)]
                    acc_b = t[0] * rn[s][rb, nrow_b, pl.ds(0, L)]
                    for j in range(1, n_sub):
                        acc_a = acc_a + t[j] * rn[s][ra, nrow_a,
                                                     pl.ds(j * L, L)]
                        acc_b = acc_b + t[j] * rn[s][rb, nrow_b,
                                                     pl.ds(j * L, L)]
                    idx = jnp.where(lane < 8, lane * 0 + (ch * CN + ra),
                                    lane * 0 + (ch * CN + rb))
                    plsc.store_scatter(negbuf, [idx], hsum2(acc_a, acc_b),
                                       mask=lane07)

        fire(0, 0)
        fire(1, 1)
        fire(2, 2)

        def quad(qq, _):
            ch0 = 4 * qq
            for p in range(4):
                @pl.when(ch0 + p + 3 < n_chunks)
                def _(p=p):
                    fire(ch0 + p + 3, (p + 3) % 4)
                compute(ch0 + p, p)
            return 0

        lax.fori_loop(0, n_chunks // 4, quad, 0)

        pltpu.sync_copy(posbuf, pos_hbm.at[pl.ds(base, PB)])
        pltpu.sync_copy(negbuf, neg_hbm.at[pl.ds(base * K, PN)])

    return sc_kernel


def _tc_loss_kernel(pos_ref, neg_ref, out_ref):
    p = pos_ref[...]
    n = neg_ref[...]
    pos_ls = jnp.minimum(p, 0.0) - jnp.log1p(jnp.exp(-jnp.abs(p)))
    m = -n
    neg_ls = jnp.minimum(m, 0.0) - jnp.log1p(jnp.exp(-jnp.abs(m)))
    total = jnp.sum(pos_ls) + jnp.sum(neg_ls)
    out_ref[0, 0] = -total / p.size


def kernel(target_ids, context_ids, neg_ids, target_W, context_W):
    B, K = neg_ids.shape
    V, D = target_W.shape
    neg_flat = neg_ids.reshape(B * K)

    sc = _sc_scores(B, K, D, C=1)
    pos_score, neg_score = sc(target_ids, context_ids, neg_flat,
                              target_W.reshape(V // 8, 8, D),
                              context_W.reshape(V // 8, 8, D))

    loss = pl.pallas_call(
        _tc_loss_kernel,
        out_shape=jax.ShapeDtypeStruct((1, 1), jnp.float32),
        out_specs=pl.BlockSpec(memory_space=pltpu.SMEM),
    )(pos_score.reshape(B // 128, 128), neg_score.reshape(B * K // 128, 128))
    return loss[0, 0]
